# Initial kernel scaffold; baseline (speedup 1.0000x reference)
import jax
import jax.numpy as jnp
from jax.experimental import pallas as pl


def kernel(feat, pre0_dst, pre0_src, pre1_dst, pre1_src, suc0_dst, suc0_src, suc1_dst, suc1_src, left_dst, left_src, right_dst, right_src, W_ctr, W_pre0, W_pre1, W_suc0, W_suc1, W_left, W_right, W_ctr2, gamma1, beta1, gamma2, beta2):
    raise NotImplementedError("write your pallas kernel here")


# speedup vs baseline: 3.4503x; 3.4503x over previous
import functools

import jax
import jax.numpy as jnp
from jax import lax
from jax.experimental import pallas as pl
from jax.experimental.pallas import tpu as pltpu
from jax.experimental.pallas import tpu_sc as plsc

N = 10000
C = 128
NREL = 6
E_REAL = 4 * 100000 + 2 * 20000
NTILES = 32
EPT = 13824
E_PAD = NTILES * EPT
CHUNK = 128
NCHUNK = EPT // CHUNK
ACC_ROWS = 10112
RPT = ACC_ROWS // 16
BLK = 1000
GRID = N // BLK


def _proj_body(x_ref, wc_ref, wr_ref, t0_ref, h_ref):
    x = x_ref[...]
    t0_ref[...] = jnp.dot(x, wc_ref[...], preferred_element_type=jnp.float32)
    for k in range(NREL):
        h_ref[k] = jnp.dot(x, wr_ref[k], preferred_element_type=jnp.float32)


_proj = pl.pallas_call(
    _proj_body,
    grid=(GRID,),
    in_specs=[
        pl.BlockSpec((BLK, C), lambda i: (i, 0)),
        pl.BlockSpec((C, C), lambda i: (0, 0)),
        pl.BlockSpec((NREL, C, C), lambda i: (0, 0, 0)),
    ],
    out_specs=[
        pl.BlockSpec((BLK, C), lambda i: (i, 0)),
        pl.BlockSpec((NREL, BLK, C), lambda i: (0, i, 0)),
    ],
    out_shape=[
        jax.ShapeDtypeStruct((N, C), jnp.float32),
        jax.ShapeDtypeStruct((NREL, N, C), jnp.float32),
    ],
)


def _sc_body(h_hbm, src_hbm, dst_hbm, out_hbm, idx_s, idx_d, rows, acc, sem):
    c = lax.axis_index("c")
    s = lax.axis_index("s")
    wid = s * 2 + c

    zero = jnp.zeros((16,), jnp.float32)

    def zloop(i, carry):
        r = i // 8
        col = (i % 8) * 16
        rows[r, pl.ds(col, 16)] = zero
        return carry

    lax.fori_loop(0, CHUNK * (C // 16), zloop, 0)

    zbase = s * RPT
    for j in range(RPT // CHUNK):
        pltpu.sync_copy(rows, acc.at[pl.ds(zbase + j * CHUNK, CHUNK)])
    rem = RPT % CHUNK
    if rem:
        pltpu.sync_copy(rows.at[pl.ds(0, rem)],
                        acc.at[pl.ds(zbase + (RPT // CHUNK) * CHUNK, rem)])
    plsc.subcore_barrier()

    ebase = wid * EPT

    def chunk_loop(ci, carry):
        off = pl.multiple_of(ebase + ci * CHUNK, CHUNK)
        pltpu.sync_copy(src_hbm.at[pl.ds(off, CHUNK)], idx_s)
        pltpu.sync_copy(dst_hbm.at[pl.ds(off, CHUNK)], idx_d)
        pltpu.async_copy(h_hbm.at[idx_s], rows, sem).wait()
        pltpu.sync_copy(rows, acc.at[idx_d], add=True)
        return carry

    lax.fori_loop(0, NCHUNK, chunk_loop, 0)
    plsc.subcore_barrier()

    pltpu.sync_copy(acc.at[pl.ds(zbase, RPT)], out_hbm.at[c, pl.ds(zbase, RPT)])


@functools.cache
def _get_sc_scatter():
    return functools.partial(
        pl.kernel,
        out_type=jax.ShapeDtypeStruct((2, ACC_ROWS, C), jnp.float32),
        mesh=plsc.VectorSubcoreMesh(core_axis_name="c", subcore_axis_name="s",
                                    num_cores=2, num_subcores=16),
        scratch_types=[
            pltpu.VMEM((CHUNK,), jnp.int32),
            pltpu.VMEM((CHUNK,), jnp.int32),
            pltpu.VMEM((CHUNK, C), jnp.float32),
            pltpu.VMEM_SHARED((ACC_ROWS, C), jnp.float32),
            pltpu.SemaphoreType.DMA,
        ],
    )(_sc_body)


def _comb_body(t0_ref, p_ref, res_ref, g1_ref, b1_ref, g2_ref, b2_ref,
               w2_ref, out_ref):
    t = t0_ref[...] + p_ref[0] + p_ref[1]
    mu = jnp.mean(t, axis=1, keepdims=True)
    var = jnp.mean((t - mu) * (t - mu), axis=1, keepdims=True)
    h = (t - mu) * lax.rsqrt(var + 1e-5) * g1_ref[...] + b1_ref[...]
    h = jnp.maximum(h, 0.0)
    h = jnp.dot(h, w2_ref[...], preferred_element_type=jnp.float32)
    mu2 = jnp.mean(h, axis=1, keepdims=True)
    var2 = jnp.mean((h - mu2) * (h - mu2), axis=1, keepdims=True)
    h = (h - mu2) * lax.rsqrt(var2 + 1e-5) * g2_ref[...] + b2_ref[...]
    out_ref[...] = jnp.maximum(h + res_ref[...], 0.0)


_combine = pl.pallas_call(
    _comb_body,
    grid=(GRID,),
    in_specs=[
        pl.BlockSpec((BLK, C), lambda i: (i, 0)),
        pl.BlockSpec((2, BLK, C), lambda i: (0, i, 0)),
        pl.BlockSpec((BLK, C), lambda i: (i, 0)),
        pl.BlockSpec((1, C), lambda i: (0, 0)),
        pl.BlockSpec((1, C), lambda i: (0, 0)),
        pl.BlockSpec((1, C), lambda i: (0, 0)),
        pl.BlockSpec((1, C), lambda i: (0, 0)),
        pl.BlockSpec((C, C), lambda i: (0, 0)),
    ],
    out_specs=pl.BlockSpec((BLK, C), lambda i: (i, 0)),
    out_shape=jax.ShapeDtypeStruct((N, C), jnp.float32),
)


def kernel(feat, pre0_dst, pre0_src, pre1_dst, pre1_src, suc0_dst, suc0_src,
           suc1_dst, suc1_src, left_dst, left_src, right_dst, right_src,
           W_ctr, W_pre0, W_pre1, W_suc0, W_suc1, W_left, W_right,
           W_ctr2, gamma1, beta1, gamma2, beta2):
    srcs = [pre0_src, pre1_src, suc0_src, suc1_src, left_src, right_src]
    dsts = [pre0_dst, pre1_dst, suc0_dst, suc1_dst, left_dst, right_dst]
    pad = E_PAD - E_REAL
    src_all = jnp.concatenate(
        [s + jnp.int32(k * N) for k, s in enumerate(srcs)]
        + [jnp.zeros((pad,), jnp.int32)])
    dst_all = jnp.concatenate(dsts + [jnp.full((pad,), N, jnp.int32)])

    wc_t = jnp.transpose(W_ctr, (0, 2, 1))
    wr_t = jnp.stack(
        [jnp.transpose(w, (0, 2, 1))
         for w in (W_pre0, W_pre1, W_suc0, W_suc1, W_left, W_right)], axis=1)
    w2_t = jnp.transpose(W_ctr2, (0, 2, 1))

    x = feat
    for i in range(4):
        t0, h = _proj(x, wc_t[i], wr_t[i])
        partials = _get_sc_scatter()(h.reshape(NREL * N, C), src_all, dst_all)
        x = _combine(t0, partials, x, gamma1[i:i + 1], beta1[i:i + 1],
                     gamma2[i:i + 1], beta2[i:i + 1], w2_t[i])
    return x


# speedup vs baseline: 5.2806x; 1.5305x over previous
import functools

import jax
import jax.numpy as jnp
from jax import lax
from jax.experimental import pallas as pl
from jax.experimental.pallas import tpu as pltpu
from jax.experimental.pallas import tpu_sc as plsc

N = 10000
C = 128
NREL = 6
E_REAL = 4 * 100000 + 2 * 20000
NTILES = 32
EPT = 13824
E_PAD = NTILES * EPT
CHUNK = 128
NCHUNK = EPT // CHUNK
ACC_ROWS = 10112
RPT = ACC_ROWS // 16
BLK = 1000
GRID = N // BLK


def _proj_body(x_ref, wc_ref, wr_ref, t0_ref, h_ref):
    x = x_ref[...]
    t0_ref[...] = jnp.dot(x, wc_ref[...], preferred_element_type=jnp.float32)
    for k in range(NREL):
        h_ref[k] = jnp.dot(x, wr_ref[k], preferred_element_type=jnp.float32)


_proj = pl.pallas_call(
    _proj_body,
    grid=(GRID,),
    in_specs=[
        pl.BlockSpec((BLK, C), lambda i: (i, 0)),
        pl.BlockSpec((C, C), lambda i: (0, 0)),
        pl.BlockSpec((NREL, C, C), lambda i: (0, 0, 0)),
    ],
    out_specs=[
        pl.BlockSpec((BLK, C), lambda i: (i, 0)),
        pl.BlockSpec((NREL, BLK, C), lambda i: (0, i, 0)),
    ],
    out_shape=[
        jax.ShapeDtypeStruct((N, C), jnp.float32),
        jax.ShapeDtypeStruct((NREL, N, C), jnp.float32),
    ],
)


def _sc_body(h_hbm, src_hbm, dst_hbm, z_hbm, out_hbm,
             idx_s, dstb0, dstb1, rows0, rows1, acc,
             sem0, sem1, semd0, semd1, semz):
    c = lax.axis_index("c")
    s = lax.axis_index("s")
    wid = s * 2 + c
    zbase = s * RPT
    ebase = wid * EPT

    zcp = pltpu.async_copy(z_hbm.at[pl.ds(zbase, RPT)],
                           acc.at[pl.ds(zbase, RPT)], semz)
    pltpu.sync_copy(src_hbm.at[wid], idx_s)

    pltpu.async_copy(dst_hbm.at[pl.ds(pl.multiple_of(ebase, CHUNK), CHUNK)],
                     dstb0, semd0)
    pltpu.async_copy(dst_hbm.at[pl.ds(pl.multiple_of(ebase + CHUNK, CHUNK),
                                      CHUNK)], dstb1, semd1)
    pltpu.async_copy(h_hbm.at[idx_s.at[0]], rows0, sem0)
    pltpu.async_copy(h_hbm.at[idx_s.at[1]], rows1, sem1)
    zcp.wait()
    plsc.subcore_barrier()

    def outer(g, carry):
        ring = ((rows0, sem0, dstb0, semd0), (rows1, sem1, dstb1, semd1))
        for b, (rows, sem, dstb, semd) in enumerate(ring):
            ci = g * 2 + b
            pltpu.make_async_copy(h_hbm.at[idx_s.at[ci]], rows, sem).wait()
            pltpu.make_async_copy(dst_hbm.at[pl.ds(0, CHUNK)], dstb,
                                  semd).wait()
            pltpu.sync_copy(rows, acc.at[dstb], add=True)

            @pl.when(ci + 2 < NCHUNK)
            def _():
                off = pl.multiple_of(ebase + (ci + 2) * CHUNK, CHUNK)
                pltpu.async_copy(dst_hbm.at[pl.ds(off, CHUNK)], dstb, semd)
                pltpu.async_copy(h_hbm.at[idx_s.at[ci + 2]], rows, sem)
        return carry

    lax.fori_loop(0, NCHUNK // 2, outer, 0)
    plsc.subcore_barrier()

    pltpu.sync_copy(acc.at[pl.ds(zbase, RPT)], out_hbm.at[c, pl.ds(zbase, RPT)])


@functools.cache
def _get_sc_scatter():
    return functools.partial(
        pl.kernel,
        out_type=jax.ShapeDtypeStruct((2, ACC_ROWS, C), jnp.float32),
        mesh=plsc.VectorSubcoreMesh(core_axis_name="c", subcore_axis_name="s",
                                    num_cores=2, num_subcores=16),
        scratch_types=[
            pltpu.VMEM((NCHUNK, CHUNK), jnp.int32),
            pltpu.VMEM((CHUNK,), jnp.int32),
            pltpu.VMEM((CHUNK,), jnp.int32),
            pltpu.VMEM((CHUNK, C), jnp.float32),
            pltpu.VMEM((CHUNK, C), jnp.float32),
            pltpu.VMEM_SHARED((ACC_ROWS, C), jnp.float32),
            pltpu.SemaphoreType.DMA,
            pltpu.SemaphoreType.DMA,
            pltpu.SemaphoreType.DMA,
            pltpu.SemaphoreType.DMA,
            pltpu.SemaphoreType.DMA,
        ],
    )(_sc_body)


def _comb_body(t0_ref, p_ref, res_ref, g1_ref, b1_ref, g2_ref, b2_ref,
               w2_ref, out_ref):
    t = t0_ref[...] + p_ref[0] + p_ref[1]
    mu = jnp.mean(t, axis=1, keepdims=True)
    var = jnp.mean((t - mu) * (t - mu), axis=1, keepdims=True)
    h = (t - mu) * lax.rsqrt(var + 1e-5) * g1_ref[...] + b1_ref[...]
    h = jnp.maximum(h, 0.0)
    h = jnp.dot(h, w2_ref[...], preferred_element_type=jnp.float32)
    mu2 = jnp.mean(h, axis=1, keepdims=True)
    var2 = jnp.mean((h - mu2) * (h - mu2), axis=1, keepdims=True)
    h = (h - mu2) * lax.rsqrt(var2 + 1e-5) * g2_ref[...] + b2_ref[...]
    out_ref[...] = jnp.maximum(h + res_ref[...], 0.0)


_combine = pl.pallas_call(
    _comb_body,
    grid=(GRID,),
    in_specs=[
        pl.BlockSpec((BLK, C), lambda i: (i, 0)),
        pl.BlockSpec((2, BLK, C), lambda i: (0, i, 0)),
        pl.BlockSpec((BLK, C), lambda i: (i, 0)),
        pl.BlockSpec((1, C), lambda i: (0, 0)),
        pl.BlockSpec((1, C), lambda i: (0, 0)),
        pl.BlockSpec((1, C), lambda i: (0, 0)),
        pl.BlockSpec((1, C), lambda i: (0, 0)),
        pl.BlockSpec((C, C), lambda i: (0, 0)),
    ],
    out_specs=pl.BlockSpec((BLK, C), lambda i: (i, 0)),
    out_shape=jax.ShapeDtypeStruct((N, C), jnp.float32),
)


def kernel(feat, pre0_dst, pre0_src, pre1_dst, pre1_src, suc0_dst, suc0_src,
           suc1_dst, suc1_src, left_dst, left_src, right_dst, right_src,
           W_ctr, W_pre0, W_pre1, W_suc0, W_suc1, W_left, W_right,
           W_ctr2, gamma1, beta1, gamma2, beta2):
    srcs = [pre0_src, pre1_src, suc0_src, suc1_src, left_src, right_src]
    dsts = [pre0_dst, pre1_dst, suc0_dst, suc1_dst, left_dst, right_dst]
    pad = E_PAD - E_REAL
    src_all = jnp.concatenate(
        [s + jnp.int32(k * N) for k, s in enumerate(srcs)]
        + [jnp.zeros((pad,), jnp.int32)]).reshape(NTILES, NCHUNK, CHUNK)
    dst_all = jnp.concatenate(dsts + [jnp.full((pad,), N, jnp.int32)])
    zeros_acc = jnp.zeros((ACC_ROWS, C), jnp.float32)

    wc_t = jnp.transpose(W_ctr, (0, 2, 1))
    wr_t = jnp.stack(
        [jnp.transpose(w, (0, 2, 1))
         for w in (W_pre0, W_pre1, W_suc0, W_suc1, W_left, W_right)], axis=1)
    w2_t = jnp.transpose(W_ctr2, (0, 2, 1))

    x = feat
    for i in range(4):
        t0, h = _proj(x, wc_t[i], wr_t[i])
        partials = _get_sc_scatter()(h.reshape(NREL * N, C), src_all, dst_all,
                                     zeros_acc)
        x = _combine(t0, partials, x, gamma1[i:i + 1], beta1[i:i + 1],
                     gamma2[i:i + 1], beta2[i:i + 1], w2_t[i])
    return x
